# baseline (speedup 1.0000x reference)
import functools

import jax
import jax.numpy as jnp
from jax.experimental import pallas as pl
from jax.experimental.pallas import tpu as pltpu


def _rollout_chunk(x0_ref, u_ref, w1x_ref, w1u_ref, b1_ref, w2_ref, b2_ref,
                   out_ref, x_carry, *, chunk, nsub):
    t = pl.program_id(1)

    @pl.when(t == 0)
    def _init():
        x_carry[...] = x0_ref[...]

    rows = x0_ref.shape[0]
    sub = rows // nsub
    w1x = w1x_ref[...]
    w1u = w1u_ref[...]
    b1 = b1_ref[...]
    w2 = w2_ref[...]
    b2 = b2_ref[...]

    def uproj(j):
        return jnp.dot(u_ref[j], w1u, preferred_element_type=jnp.float32) + b1

    xs = [x_carry[s * sub:(s + 1) * sub, :] for s in range(nsub)]
    up_next = uproj(0)
    for j in range(chunk):
        up_cur = up_next
        if j + 1 < chunk:
            up_next = uproj(j + 1)
        xcat = xs[0] if nsub == 1 else jnp.concatenate(xs, axis=0)
        pre = jnp.dot(xcat, w1x, preferred_element_type=jnp.float32)
        for s in range(nsub):
            h = jnp.tanh(pre[s * sub:(s + 1) * sub, :]
                         + up_cur[s * sub:(s + 1) * sub, :])
            dx = jnp.dot(h, w2, preferred_element_type=jnp.float32)
            xs[s] = xs[s] + dx[:, :x0_ref.shape[1]] + b2
            out_ref[j, s * sub:(s + 1) * sub, :] = xs[s]
    for s in range(nsub):
        x_carry[s * sub:(s + 1) * sub, :] = xs[s]


def kernel(x0, u, w1x, w1u, b1, w2, b2, dt):
    N, B, NU = u.shape
    _, NX = x0.shape
    H = w1x.shape[1]

    if N == 0:
        return jnp.zeros((0, B, NX), jnp.float32)

    chunk = 64
    btile = 256
    nsub = 2

    NXP = max(256, NX)
    w2dt = jnp.zeros((H, NXP), jnp.float32).at[:, :NX].set(
        (w2 * dt).astype(jnp.float32))
    b2dt = (b2 * dt).astype(jnp.float32)

    B_pad = ((B + btile - 1) // btile) * btile
    N_pad = ((N + chunk - 1) // chunk) * chunk
    x0_p = jnp.zeros((B_pad, NX), jnp.float32).at[:B].set(x0.astype(jnp.float32))
    u_p = jnp.zeros((N_pad, B_pad, NU), jnp.float32).at[:N, :B].set(
        u.astype(jnp.float32))

    grid = (B_pad // btile, N_pad // chunk)

    out = pl.pallas_call(
        functools.partial(_rollout_chunk, chunk=chunk, nsub=nsub),
        out_shape=jax.ShapeDtypeStruct((N_pad, B_pad, NX), jnp.float32),
        grid=grid,
        in_specs=[
            pl.BlockSpec((btile, NX), lambda b, t: (b, 0)),
            pl.BlockSpec((chunk, btile, NU), lambda b, t: (t, b, 0)),
            pl.BlockSpec((NX, H), lambda b, t: (0, 0)),
            pl.BlockSpec((NU, H), lambda b, t: (0, 0)),
            pl.BlockSpec((1, H), lambda b, t: (0, 0)),
            pl.BlockSpec((H, NXP), lambda b, t: (0, 0)),
            pl.BlockSpec((1, NX), lambda b, t: (0, 0)),
        ],
        out_specs=pl.BlockSpec((chunk, btile, NX), lambda b, t: (t, b, 0)),
        scratch_shapes=[pltpu.VMEM((btile, NX), jnp.float32)],
        compiler_params=pltpu.CompilerParams(
            dimension_semantics=("parallel", "arbitrary")),
    )(x0_p, u_p, w1x.astype(jnp.float32), w1u.astype(jnp.float32),
      b1.astype(jnp.float32), w2dt, b2dt)

    return out[:N, :B, :]


# speedup vs baseline: 1.0244x; 1.0244x over previous
import functools

import jax
import jax.numpy as jnp
from jax.experimental import pallas as pl
from jax.experimental.pallas import tpu as pltpu


def _rollout_chunk(x0_ref, u_ref, w1x_ref, w1u_ref, b1_ref, w2_ref, b2_ref,
                   out_ref, x_carry, *, chunk, nsub):
    t = pl.program_id(1)

    @pl.when(t == 0)
    def _init():
        x_carry[...] = x0_ref[...]

    rows = x0_ref.shape[0]
    sub = rows // nsub
    w1x = w1x_ref[...]
    w1u = w1u_ref[...]
    b1 = b1_ref[...]
    w2 = w2_ref[...]
    b2 = b2_ref[...]

    def uproj(j):
        return jnp.dot(u_ref[j], w1u, preferred_element_type=jnp.float32) + b1

    xs = [x_carry[s * sub:(s + 1) * sub, :] for s in range(nsub)]
    up_next = uproj(0)
    for j in range(chunk):
        up_cur = up_next
        if j + 1 < chunk:
            up_next = uproj(j + 1)
        xcat = xs[0] if nsub == 1 else jnp.concatenate(xs, axis=0)
        pre = jnp.dot(xcat, w1x, preferred_element_type=jnp.float32)
        for s in range(nsub):
            h = jnp.tanh(pre[s * sub:(s + 1) * sub, :]
                         + up_cur[s * sub:(s + 1) * sub, :])
            xs[s] = (xs[s] + jnp.dot(h, w2, preferred_element_type=jnp.float32)
                     + b2)
            out_ref[j, s * sub:(s + 1) * sub, :] = xs[s]
    for s in range(nsub):
        x_carry[s * sub:(s + 1) * sub, :] = xs[s]


def kernel(x0, u, w1x, w1u, b1, w2, b2, dt):
    N, B, NU = u.shape
    _, NX = x0.shape
    H = w1x.shape[1]

    if N == 0:
        return jnp.zeros((0, B, NX), jnp.float32)

    chunk = 64
    btile = 256
    nsub = 2

    w2dt = (w2 * dt).astype(jnp.float32)
    b2dt = (b2 * dt).astype(jnp.float32)

    B_pad = ((B + btile - 1) // btile) * btile
    N_pad = ((N + chunk - 1) // chunk) * chunk
    x0_p = jnp.zeros((B_pad, NX), jnp.float32).at[:B].set(x0.astype(jnp.float32))
    u_p = jnp.zeros((N_pad, B_pad, NU), jnp.float32).at[:N, :B].set(
        u.astype(jnp.float32))

    grid = (B_pad // btile, N_pad // chunk)

    out = pl.pallas_call(
        functools.partial(_rollout_chunk, chunk=chunk, nsub=nsub),
        out_shape=jax.ShapeDtypeStruct((N_pad, B_pad, NX), jnp.float32),
        grid=grid,
        in_specs=[
            pl.BlockSpec((btile, NX), lambda b, t: (b, 0)),
            pl.BlockSpec((chunk, btile, NU), lambda b, t: (t, b, 0)),
            pl.BlockSpec((NX, H), lambda b, t: (0, 0)),
            pl.BlockSpec((NU, H), lambda b, t: (0, 0)),
            pl.BlockSpec((1, H), lambda b, t: (0, 0)),
            pl.BlockSpec((H, NX), lambda b, t: (0, 0)),
            pl.BlockSpec((1, NX), lambda b, t: (0, 0)),
        ],
        out_specs=pl.BlockSpec((chunk, btile, NX), lambda b, t: (t, b, 0)),
        scratch_shapes=[pltpu.VMEM((btile, NX), jnp.float32)],
        compiler_params=pltpu.CompilerParams(
            dimension_semantics=("parallel", "arbitrary")),
    )(x0_p, u_p, w1x.astype(jnp.float32), w1u.astype(jnp.float32),
      b1.astype(jnp.float32), w2dt, b2dt)

    return out[:N, :B, :]
